# baseline (speedup 1.0000x reference)
import math

import jax
import jax.numpy as jnp
from jax.experimental import pallas as pl
from jax.experimental.pallas import tpu as pltpu

_B, _S, _D, _NSENT, _NL = 8, 2048, 1024, 32, 2
_NPAD = 40
_BN = _B * _NPAD

_INTERPRET = False


def _pool_body(ind_ref, tok_ref, gv_ref):
    ind = ind_ref[0]
    tok = tok_ref[0]
    sent = jax.lax.broadcasted_iota(jnp.int32, (_NSENT, _S), 0)
    oh = (ind == sent).astype(jnp.float32)
    counts = jnp.sum(oh, axis=1, keepdims=True)
    sums = jax.lax.dot_general(oh, tok, (((1,), (0,)), ((), ())),
                               preferred_element_type=jnp.float32)
    node0 = tok[0:1, :]
    node1 = (sums[0:1] - node0) / jnp.maximum(counts[0:1] - 1.0, 1.0)
    means = sums[1:] / jnp.maximum(counts[1:], 1.0)
    pad = jnp.zeros((_NPAD - _NSENT - 1, _D), jnp.float32)
    gv_ref[0] = jnp.concatenate([node0, node1, means, pad], axis=0)


def _pool(sent3, start_layer):
    return pl.pallas_call(
        _pool_body,
        grid=(_B,),
        in_specs=[
            pl.BlockSpec((1, 1, _S), lambda b: (b, 0, 0)),
            pl.BlockSpec((1, _S, _D), lambda b: (b, 0, 0)),
        ],
        out_specs=pl.BlockSpec((1, _NPAD, _D), lambda b: (b, 0, 0)),
        out_shape=jax.ShapeDtypeStruct((_B, _NPAD, _D), jnp.float32),
        interpret=_INTERPRET,
    )(sent3, start_layer)


def _layers_body(gv0_ref, tok_ref, wq_ref, wkt_ref, wv_ref, wo_ref,
                 g_ref, b_ref, out_ref, gv_scr, q2_scr, acc_scr, l_scr):
    i = pl.program_id(0)
    p = jax.lax.rem(i, 10)

    @pl.when(p == 0)
    def _():
        @pl.when(i == 0)
        def _():
            gv_scr[...] = gv0_ref[...].reshape(_BN, _D)
        gvm = gv_scr[...]
        q1 = jnp.dot(gvm.astype(jnp.bfloat16), wq_ref[0],
                     preferred_element_type=jnp.float32)
        q2_scr[...] = jnp.dot(q1.astype(jnp.bfloat16), wkt_ref[0],
                              preferred_element_type=jnp.float32)

    @pl.when((p >= 1) & (p <= 8))
    def _():
        b = p - 1
        tok = tok_ref[0, 0]
        tokb = tok.astype(jnp.bfloat16)
        q2 = q2_scr[pl.ds(b * _NPAD, _NPAD), :]
        scores = jax.lax.dot_general(
            q2.astype(jnp.bfloat16), tokb, (((1,), (1,)), ((), ())),
            preferred_element_type=jnp.float32) * (1.0 / math.sqrt(_D))
        m = jnp.max(scores, axis=1, keepdims=True)
        pe = jnp.exp(scores - m)
        lsum = jnp.sum(pe, axis=1, keepdims=True)
        acc = jnp.dot(pe.astype(jnp.bfloat16), tokb,
                      preferred_element_type=jnp.float32)
        acc_scr[pl.ds(b * _NPAD, _NPAD), :] = acc
        l_scr[pl.ds(b * _NPAD, _NPAD), :] = lsum

    @pl.when(p == 9)
    def _():
        u = acc_scr[...] / l_scr[...]
        o1 = jnp.dot(u.astype(jnp.bfloat16), wv_ref[0],
                     preferred_element_type=jnp.float32)
        o2 = jnp.dot(o1.astype(jnp.bfloat16), wo_ref[0],
                     preferred_element_type=jnp.float32)
        x = gv_scr[...] + o2
        mu = jnp.mean(x, axis=1, keepdims=True)
        var = jnp.mean(jnp.square(x - mu), axis=1, keepdims=True)
        y = (x - mu) * jax.lax.rsqrt(var + 1e-5) * g_ref[0] + b_ref[0]
        gv_scr[...] = y

        @pl.when(i == 10 * _NL - 1)
        def _():
            out_ref[...] = y.reshape(_B, _NPAD, _D)


def _layers(gv0, subsequent_layers, wq, wkt, wv, wo, g2, b2):
    return pl.pallas_call(
        _layers_body,
        grid=(10 * _NL,),
        in_specs=[
            pl.BlockSpec((_B, _NPAD, _D), lambda i: (0, 0, 0)),
            pl.BlockSpec((1, 1, _S, _D),
                         lambda i: (i // 10,
                                    jnp.clip(jax.lax.rem(i, 10) - 1, 0, _B - 1),
                                    0, 0)),
            pl.BlockSpec((1, _D, _D), lambda i: (i // 10, 0, 0)),
            pl.BlockSpec((1, _D, _D), lambda i: (i // 10, 0, 0)),
            pl.BlockSpec((1, _D, _D), lambda i: (i // 10, 0, 0)),
            pl.BlockSpec((1, _D, _D), lambda i: (i // 10, 0, 0)),
            pl.BlockSpec((1, 1, _D), lambda i: (i // 10, 0, 0)),
            pl.BlockSpec((1, 1, _D), lambda i: (i // 10, 0, 0)),
        ],
        out_specs=pl.BlockSpec((_B, _NPAD, _D), lambda i: (0, 0, 0)),
        out_shape=jax.ShapeDtypeStruct((_B, _NPAD, _D), jnp.float32),
        scratch_shapes=[
            pltpu.VMEM((_BN, _D), jnp.float32),
            pltpu.VMEM((_BN, _D), jnp.float32),
            pltpu.VMEM((_BN, _D), jnp.float32),
            pltpu.VMEM((_BN, 1), jnp.float32),
        ],
        interpret=_INTERPRET,
    )(gv0, subsequent_layers, wq, wkt, wv, wo, g2, b2)


def kernel(sent_ind, start_layer, subsequent_layers, Wq, Wk, Wv, Wo, ln_g, ln_b):
    sent3 = sent_ind.reshape(_B, 1, _S)
    gv0 = _pool(sent3, start_layer)
    wq = Wq.astype(jnp.bfloat16)
    wkt = jnp.swapaxes(Wk, 1, 2).astype(jnp.bfloat16)
    wv = Wv.astype(jnp.bfloat16)
    wo = Wo.astype(jnp.bfloat16)
    g2 = ln_g.reshape(_NL, 1, _D)
    b2 = ln_b.reshape(_NL, 1, _D)
    gv = _layers(gv0, subsequent_layers, wq, wkt, wv, wo, g2, b2)
    return gv[:, :33, :]
